# Initial kernel scaffold; baseline (speedup 1.0000x reference)
import jax
import jax.numpy as jnp
from jax.experimental import pallas as pl


def kernel(x, edge_index, W1, b1, W2, b2, W3, b3, W4, b4, Wm1, bm1, Wm2, bm2, Wm3, bm3):
    raise NotImplementedError("write your pallas kernel here")


# speedup vs baseline: 14.1682x; 14.1682x over previous
import functools

import jax
import jax.numpy as jnp
from jax import lax
from jax.experimental import pallas as pl
from jax.experimental.pallas import tpu as pltpu
from jax.experimental.pallas import tpu_sc as plsc

N = 10000
D = 128
H = 128
C = 64
E = 320000

NC = 2
NS = 16
NW = NC * NS
EPW = E // NW
CHUNK = 80
NCHUNK = EPW // CHUNK
RCH = 80
NRCH = N // RCH
RITER = -(-NRCH // NS)
DEG_W = 128


def _strided_rows(s, fn):
    def body(j, carry):
        idx = s + NS * j

        @pl.when(idx < NRCH)
        def _():
            fn(idx * RCH)

        return carry

    lax.fori_loop(0, RITER, body, 0)

_mesh = plsc.VectorSubcoreMesh(core_axis_name="c", subcore_axis_name="s")


@functools.partial(
    pl.kernel,
    out_type=jax.ShapeDtypeStruct((NC, N, DEG_W), jnp.float32),
    mesh=_mesh,
    scratch_types=[
        pltpu.VMEM((NCHUNK, CHUNK), jnp.int32),
        pltpu.VMEM((CHUNK, DEG_W), jnp.float32),
        pltpu.VMEM_SHARED((N, DEG_W), jnp.float32),
    ],
)
def _deg_kernel(dst_hbm, zeros_hbm, ones_hbm, out_hbm, idx_v, ones_v, deg_sh):
    c = lax.axis_index("c")
    s = lax.axis_index("s")
    wid = s * NC + c
    _strided_rows(s, lambda r: pltpu.sync_copy(
        zeros_hbm.at[pl.ds(r, RCH)], deg_sh.at[pl.ds(r, RCH)]))
    pltpu.sync_copy(dst_hbm.at[wid], idx_v)
    pltpu.sync_copy(ones_hbm, ones_v)
    plsc.subcore_barrier()

    def body(i, carry):
        pltpu.sync_copy(ones_v, deg_sh.at[idx_v.at[i]], add=True)
        return carry

    lax.fori_loop(0, NCHUNK, body, 0)
    plsc.subcore_barrier()
    _strided_rows(s, lambda r: pltpu.sync_copy(
        deg_sh.at[pl.ds(r, RCH)], out_hbm.at[c, pl.ds(r, RCH)]))


@functools.partial(
    pl.kernel,
    out_type=jax.ShapeDtypeStruct((NC, N, D), jnp.float32),
    mesh=_mesh,
    scratch_types=[
        pltpu.VMEM((NCHUNK, CHUNK), jnp.int32),
        pltpu.VMEM((NCHUNK, CHUNK), jnp.int32),
        pltpu.VMEM((CHUNK, D), jnp.float32),
        pltpu.VMEM_SHARED((N, D), jnp.float32),
        pltpu.SemaphoreType.DMA,
    ],
)
def _agg_kernel(p_hbm, src_hbm, dst_hbm, zeros_hbm, out_hbm,
                sidx_v, didx_v, rows_v, q_sh, gsem):
    c = lax.axis_index("c")
    s = lax.axis_index("s")
    wid = s * NC + c
    _strided_rows(s, lambda r: pltpu.sync_copy(
        zeros_hbm.at[pl.ds(r, RCH)], q_sh.at[pl.ds(r, RCH)]))
    pltpu.sync_copy(src_hbm.at[wid], sidx_v)
    pltpu.sync_copy(dst_hbm.at[wid], didx_v)
    plsc.subcore_barrier()

    def body(i, carry):
        pltpu.async_copy(p_hbm.at[sidx_v.at[i]], rows_v, gsem).wait()
        pltpu.sync_copy(rows_v, q_sh.at[didx_v.at[i]], add=True)
        return carry

    lax.fori_loop(0, NCHUNK, body, 0)
    plsc.subcore_barrier()
    _strided_rows(s, lambda r: pltpu.sync_copy(
        q_sh.at[pl.ds(r, RCH)], out_hbm.at[c, pl.ds(r, RCH)]))


ROWS = 2000
GRID = N // ROWS


def _dinv(deg_ref):
    d = deg_ref[0, :, 0:1] + deg_ref[1, :, 0:1] + 1.0
    return lax.rsqrt(d)


def _elu(v):
    return jnp.where(v > 0, v, jnp.exp(jnp.minimum(v, 0.0)) - 1.0)


def _prep_body(deg_ref, x_ref, w_ref, p_ref):
    dinv = _dinv(deg_ref)
    u = jnp.dot(x_ref[...], w_ref[...], preferred_element_type=jnp.float32)
    p_ref[...] = u * dinv


def _mid_body(deg_ref, q_ref, p_ref, b_ref, w_ref, out_ref):
    dinv = _dinv(deg_ref)
    h = _elu((q_ref[0] + q_ref[1] + p_ref[...]) * dinv + b_ref[...])
    out_ref[...] = jnp.dot(
        h, w_ref[...], preferred_element_type=jnp.float32) * dinv


def _fin_body(deg_ref, q_ref, p_ref, b_ref, wm_ref, bm_ref, h_ref, s_ref):
    dinv = _dinv(deg_ref)
    h = _elu((q_ref[0] + q_ref[1] + p_ref[...]) * dinv + b_ref[...])
    h_ref[...] = h
    s_ref[...] = _elu(
        jnp.dot(h, wm_ref[...], preferred_element_type=jnp.float32)
        + bm_ref[...])


_deg_spec = pl.BlockSpec((2, ROWS, DEG_W), lambda i: (0, i, 0))
_row_spec = pl.BlockSpec((ROWS, D), lambda i: (i, 0))
_q_spec = pl.BlockSpec((2, ROWS, D), lambda i: (0, i, 0))
_w_spec = pl.BlockSpec((D, H), lambda i: (0, 0))
_b_spec = pl.BlockSpec((1, H), lambda i: (0, 0))

_prep_call = pl.pallas_call(
    _prep_body,
    grid=(GRID,),
    in_specs=[_deg_spec, _row_spec, _w_spec],
    out_specs=_row_spec,
    out_shape=jax.ShapeDtypeStruct((N, H), jnp.float32),
)

_mid_call = pl.pallas_call(
    _mid_body,
    grid=(GRID,),
    in_specs=[_deg_spec, _q_spec, _row_spec, _b_spec, _w_spec],
    out_specs=_row_spec,
    out_shape=jax.ShapeDtypeStruct((N, H), jnp.float32),
)

_fin_call = pl.pallas_call(
    _fin_body,
    grid=(GRID,),
    in_specs=[_deg_spec, _q_spec, _row_spec, _b_spec,
              pl.BlockSpec((H, C), lambda i: (0, 0)),
              pl.BlockSpec((1, C), lambda i: (0, 0))],
    out_specs=[_row_spec, pl.BlockSpec((ROWS, C), lambda i: (i, 0))],
    out_shape=[jax.ShapeDtypeStruct((N, H), jnp.float32),
               jax.ShapeDtypeStruct((N, C), jnp.float32)],
)


def kernel(x, edge_index, W1, b1, W2, b2, W3, b3, W4, b4,
           Wm1, bm1, Wm2, bm2, Wm3, bm3):
    ei = edge_index.astype(jnp.int32)
    src = ei[0].reshape(NW, NCHUNK, CHUNK)
    dst = ei[1].reshape(NW, NCHUNK, CHUNK)
    zeros_d = jnp.zeros((N, D), jnp.float32)
    zeros_w = jnp.zeros((N, DEG_W), jnp.float32)
    ones_w = jnp.ones((CHUNK, DEG_W), jnp.float32)

    deg2 = _deg_kernel(dst, zeros_w, ones_w)
    p = _prep_call(deg2, x, W1)
    for b, Wn in ((b1, W2), (b2, W3), (b3, W4)):
        q = _agg_kernel(p, src, dst, zeros_d)
        p = _mid_call(deg2, q, p, b.reshape(1, H), Wn)
    q = _agg_kernel(p, src, dst, zeros_d)
    h, S = _fin_call(deg2, q, p, b4.reshape(1, H), Wm3, bm3.reshape(1, C))
    return (h, S)


# speedup vs baseline: 21.1452x; 1.4924x over previous
import functools

import jax
import jax.numpy as jnp
from jax import lax
from jax.experimental import pallas as pl
from jax.experimental.pallas import tpu as pltpu
from jax.experimental.pallas import tpu_sc as plsc

N = 10000
D = 128
H = 128
C = 64
E = 320000

NC = 2
NS = 16
NW = NC * NS
EPW = E // NW
CHUNK = 80
NCHUNK = EPW // CHUNK
RCH = 80
NRCH = N // RCH
RITER = -(-NRCH // NS)
DEG_W = 128


def _strided_rows(s, fn):
    def body(j, carry):
        idx = s + NS * j

        @pl.when(idx < NRCH)
        def _():
            fn(idx * RCH)

        return carry

    lax.fori_loop(0, RITER, body, 0)

_mesh = plsc.VectorSubcoreMesh(core_axis_name="c", subcore_axis_name="s")


@functools.partial(
    pl.kernel,
    out_type=jax.ShapeDtypeStruct((NC, N, DEG_W), jnp.float32),
    mesh=_mesh,
    scratch_types=[
        pltpu.VMEM((NCHUNK, CHUNK), jnp.int32),
        pltpu.VMEM((CHUNK, DEG_W), jnp.float32),
        pltpu.VMEM_SHARED((N, DEG_W), jnp.float32),
        pltpu.SemaphoreType.DMA,
        pltpu.SemaphoreType.DMA,
    ],
)
def _deg_kernel(dst_hbm, zeros_hbm, ones_hbm, out_hbm, idx_v, ones_v, deg_sh,
                sem_a, sem_b):
    c = lax.axis_index("c")
    s = lax.axis_index("s")
    wid = s * NC + c
    _strided_rows(s, lambda r: pltpu.sync_copy(
        zeros_hbm.at[pl.ds(r, RCH)], deg_sh.at[pl.ds(r, RCH)]))
    pltpu.sync_copy(dst_hbm.at[wid], idx_v)
    pltpu.sync_copy(ones_hbm, ones_v)
    plsc.subcore_barrier()

    def step(i, sem):
        @pl.when(i >= 2)
        def _():
            pltpu.make_async_copy(
                ones_v, deg_sh.at[idx_v.at[i]], sem).wait()

        pltpu.async_copy(ones_v, deg_sh.at[idx_v.at[i]], sem, add=True)

    step(0, sem_a)

    def body(g, carry):
        step(2 * g + 1, sem_b)
        step(2 * g + 2, sem_a)
        return carry

    lax.fori_loop(0, (NCHUNK - 1) // 2, body, 0)
    pltpu.make_async_copy(ones_v, deg_sh.at[idx_v.at[0]], sem_a).wait()
    pltpu.make_async_copy(ones_v, deg_sh.at[idx_v.at[0]], sem_b).wait()
    plsc.subcore_barrier()
    _strided_rows(s, lambda r: pltpu.sync_copy(
        deg_sh.at[pl.ds(r, RCH)], out_hbm.at[c, pl.ds(r, RCH)]))


@functools.partial(
    pl.kernel,
    out_type=jax.ShapeDtypeStruct((NC, N, D), jnp.float32),
    mesh=_mesh,
    scratch_types=[
        pltpu.VMEM((NCHUNK, CHUNK), jnp.int32),
        pltpu.VMEM((CHUNK,), jnp.int32),
        pltpu.VMEM((CHUNK,), jnp.int32),
        pltpu.VMEM((CHUNK, D), jnp.float32),
        pltpu.VMEM((CHUNK, D), jnp.float32),
        pltpu.VMEM_SHARED((N, D), jnp.float32),
        pltpu.SemaphoreType.DMA,
        pltpu.SemaphoreType.DMA,
        pltpu.SemaphoreType.DMA,
        pltpu.SemaphoreType.DMA,
        pltpu.SemaphoreType.DMA,
        pltpu.SemaphoreType.DMA,
    ],
)
def _agg_kernel(p_hbm, src_hbm, dst_hbm, zeros_hbm, out_hbm,
                didx_v, si_a, si_b, rows_a, rows_b, q_sh,
                gsem_a, gsem_b, ssem_a, ssem_b, isem_a, isem_b):
    c = lax.axis_index("c")
    s = lax.axis_index("s")
    wid = s * NC + c
    _strided_rows(s, lambda r: pltpu.sync_copy(
        zeros_hbm.at[pl.ds(r, RCH)], q_sh.at[pl.ds(r, RCH)]))
    pltpu.sync_copy(dst_hbm.at[wid], didx_v)
    plsc.subcore_barrier()

    def step(i, si, rows, gsem, ssem, isem,
             si_o, rows_o, gsem_o, ssem_o, isem_o):
        @pl.when(i + 1 < NCHUNK)
        def _():
            @pl.when(i >= 1)
            def _():
                pltpu.make_async_copy(
                    rows_o, q_sh.at[didx_v.at[i]], ssem_o).wait()

            pltpu.make_async_copy(src_hbm.at[wid, i + 1], si_o, isem_o).wait()
            pltpu.async_copy(p_hbm.at[si_o], rows_o, gsem_o)

        pltpu.make_async_copy(p_hbm.at[si], rows, gsem).wait()

        @pl.when(i + 2 < NCHUNK)
        def _():
            pltpu.async_copy(src_hbm.at[wid, i + 2], si, isem)

        pltpu.async_copy(rows, q_sh.at[didx_v.at[i]], ssem, add=True)

    a_args = (si_a, rows_a, gsem_a, ssem_a, isem_a)
    b_args = (si_b, rows_b, gsem_b, ssem_b, isem_b)
    pltpu.async_copy(src_hbm.at[wid, 0], si_a, isem_a)
    pltpu.async_copy(src_hbm.at[wid, 1], si_b, isem_b)
    pltpu.make_async_copy(src_hbm.at[wid, 0], si_a, isem_a).wait()
    pltpu.async_copy(p_hbm.at[si_a], rows_a, gsem_a)
    step(0, *a_args, *b_args)

    def body(g, carry):
        step(2 * g + 1, *b_args, *a_args)
        step(2 * g + 2, *a_args, *b_args)
        return carry

    lax.fori_loop(0, (NCHUNK - 1) // 2, body, 0)
    pltpu.make_async_copy(rows_a, q_sh.at[didx_v.at[0]], ssem_a).wait()
    pltpu.make_async_copy(rows_b, q_sh.at[didx_v.at[0]], ssem_b).wait()
    plsc.subcore_barrier()
    _strided_rows(s, lambda r: pltpu.sync_copy(
        q_sh.at[pl.ds(r, RCH)], out_hbm.at[c, pl.ds(r, RCH)]))


ROWS = 2000
GRID = N // ROWS


def _dinv(deg_ref):
    d = deg_ref[0, :, 0:1] + deg_ref[1, :, 0:1] + 1.0
    return lax.rsqrt(d)


def _elu(v):
    return jnp.where(v > 0, v, jnp.exp(jnp.minimum(v, 0.0)) - 1.0)


def _prep_body(deg_ref, x_ref, w_ref, p_ref):
    dinv = _dinv(deg_ref)
    u = jnp.dot(x_ref[...], w_ref[...], preferred_element_type=jnp.float32)
    p_ref[...] = u * dinv


def _mid_body(deg_ref, q_ref, p_ref, b_ref, w_ref, out_ref):
    dinv = _dinv(deg_ref)
    h = _elu((q_ref[0] + q_ref[1] + p_ref[...]) * dinv + b_ref[...])
    out_ref[...] = jnp.dot(
        h, w_ref[...], preferred_element_type=jnp.float32) * dinv


def _fin_body(deg_ref, q_ref, p_ref, b_ref, wm_ref, bm_ref, h_ref, s_ref):
    dinv = _dinv(deg_ref)
    h = _elu((q_ref[0] + q_ref[1] + p_ref[...]) * dinv + b_ref[...])
    h_ref[...] = h
    s_ref[...] = _elu(
        jnp.dot(h, wm_ref[...], preferred_element_type=jnp.float32)
        + bm_ref[...])


_deg_spec = pl.BlockSpec((2, ROWS, DEG_W), lambda i: (0, i, 0))
_row_spec = pl.BlockSpec((ROWS, D), lambda i: (i, 0))
_q_spec = pl.BlockSpec((2, ROWS, D), lambda i: (0, i, 0))
_w_spec = pl.BlockSpec((D, H), lambda i: (0, 0))
_b_spec = pl.BlockSpec((1, H), lambda i: (0, 0))

_prep_call = pl.pallas_call(
    _prep_body,
    grid=(GRID,),
    in_specs=[_deg_spec, _row_spec, _w_spec],
    out_specs=_row_spec,
    out_shape=jax.ShapeDtypeStruct((N, H), jnp.float32),
)

_mid_call = pl.pallas_call(
    _mid_body,
    grid=(GRID,),
    in_specs=[_deg_spec, _q_spec, _row_spec, _b_spec, _w_spec],
    out_specs=_row_spec,
    out_shape=jax.ShapeDtypeStruct((N, H), jnp.float32),
)

_fin_call = pl.pallas_call(
    _fin_body,
    grid=(GRID,),
    in_specs=[_deg_spec, _q_spec, _row_spec, _b_spec,
              pl.BlockSpec((H, C), lambda i: (0, 0)),
              pl.BlockSpec((1, C), lambda i: (0, 0))],
    out_specs=[_row_spec, pl.BlockSpec((ROWS, C), lambda i: (i, 0))],
    out_shape=[jax.ShapeDtypeStruct((N, H), jnp.float32),
               jax.ShapeDtypeStruct((N, C), jnp.float32)],
)


def kernel(x, edge_index, W1, b1, W2, b2, W3, b3, W4, b4,
           Wm1, bm1, Wm2, bm2, Wm3, bm3):
    ei = edge_index.astype(jnp.int32)
    src = ei[0].reshape(NW, NCHUNK, CHUNK)
    dst = ei[1].reshape(NW, NCHUNK, CHUNK)
    zeros_d = jnp.zeros((N, D), jnp.float32)
    zeros_w = jnp.zeros((N, DEG_W), jnp.float32)
    ones_w = jnp.ones((CHUNK, DEG_W), jnp.float32)

    deg2 = _deg_kernel(dst, zeros_w, ones_w)
    p = _prep_call(deg2, x, W1)
    for b, Wn in ((b1, W2), (b2, W3), (b3, W4)):
        q = _agg_kernel(p, src, dst, zeros_d)
        p = _mid_call(deg2, q, p, b.reshape(1, H), Wn)
    q = _agg_kernel(p, src, dst, zeros_d)
    h, S = _fin_call(deg2, q, p, b4.reshape(1, H), Wm3, bm3.reshape(1, C))
    return (h, S)
